# Initial kernel scaffold; baseline (speedup 1.0000x reference)
import jax
import jax.numpy as jnp
from jax.experimental import pallas as pl


def kernel(x, edge_index, w1, b1, w2, b2, w_ih_l0, w_hh_l0, b_ih_l0, b_hh_l0, w_ih_l0_rev, w_hh_l0_rev, b_ih_l0_rev, b_hh_l0_rev, w_ih_l1, w_hh_l1, b_ih_l1, b_hh_l1, w_ih_l1_rev, w_hh_l1_rev, b_ih_l1_rev, b_hh_l1_rev, w_out, b_out):
    raise NotImplementedError("write your pallas kernel here")


# speedup vs baseline: 4.8297x; 4.8297x over previous
import functools

import jax
import jax.numpy as jnp
from jax import lax
from jax.experimental import pallas as pl
from jax.experimental.pallas import tpu as pltpu
from jax.experimental.pallas import tpu_sc as plsc

F32 = jnp.float32
_PREC = lax.Precision.HIGHEST


def _sc_edge_scatter(g, src, dst):
    N, H = g.shape
    E = src.shape[0]
    NC, NS = 2, 16
    NW = NC * NS
    assert E % NW == 0 and H % 16 == 0 and N % NS == 0
    EW = E // NW
    B = 128
    nfull, tail = divmod(EW, B)
    RPT = -(-(N // NS) // 8) * 8
    NPAD = RPT * NS

    mesh = plsc.VectorSubcoreMesh(core_axis_name="c", subcore_axis_name="s")
    scratch = [
        pltpu.VMEM((B,), jnp.int32),
        pltpu.VMEM((B,), jnp.int32),
        pltpu.VMEM((B, H), F32),
        pltpu.VMEM_SHARED((NPAD, H), F32),
        pltpu.VMEM((RPT, H), F32),
        pltpu.SemaphoreType.DMA,
    ]
    if tail:
        scratch += [
            pltpu.VMEM((tail,), jnp.int32),
            pltpu.VMEM((tail,), jnp.int32),
            pltpu.VMEM((tail, H), F32),
        ]

    @functools.partial(
        pl.kernel,
        out_type=jax.ShapeDtypeStruct((NC, NPAD, H), F32),
        mesh=mesh,
        scratch_types=scratch,
        compiler_params=pltpu.CompilerParams(use_tc_tiling_on_sc=False),
    )
    def k(g_hbm, src_hbm, dst_hbm, out_hbm, si, di, rows, acc, stage, sem,
          *tailrefs):
        c = lax.axis_index("c")
        s = lax.axis_index("s")
        w = s * NC + c

        def zrow(i, carry):
            for j in range(H // 16):
                stage[i, pl.ds(16 * j, 16)] = jnp.zeros((16,), F32)
            return carry
        lax.fori_loop(0, RPT, zrow, 0)
        pltpu.sync_copy(stage, acc.at[pl.ds(s * RPT, RPT)])
        plsc.subcore_barrier()

        base_w = w * EW

        def blk(i, carry):
            base = base_w + i * B
            pltpu.sync_copy(src_hbm.at[pl.ds(base, B)], si)
            pltpu.sync_copy(dst_hbm.at[pl.ds(base, B)], di)
            pltpu.async_copy(g_hbm.at[si], rows, sem).wait()
            pltpu.sync_copy(rows, acc.at[di], add=True)
            return carry
        lax.fori_loop(0, nfull, blk, 0)

        if tail:
            sit, dit, rowst = tailrefs
            base = base_w + nfull * B
            pltpu.sync_copy(src_hbm.at[pl.ds(base, tail)], sit)
            pltpu.sync_copy(dst_hbm.at[pl.ds(base, tail)], dit)
            pltpu.async_copy(g_hbm.at[sit], rowst, sem).wait()
            pltpu.sync_copy(rowst, acc.at[dit], add=True)

        plsc.subcore_barrier()
        pltpu.sync_copy(acc.at[pl.ds(s * RPT, RPT)],
                        out_hbm.at[c, pl.ds(s * RPT, RPT)])

    return k(g, src, dst)


_BR = 2000


def _dot(a, b):
    return jnp.dot(a, b, preferred_element_type=F32, precision=_PREC)


def _pre_body(degp, x, w1, dinv_o, g1_o):
    d = degp[0, :, 0:1] + degp[1, :, 0:1] + 1.0
    dinv = lax.rsqrt(d)
    dinv_o[...] = dinv
    g1_o[...] = dinv * _dot(x[...], w1[...])


def _mid_body(sp, g, dinv, b, w, out_o):
    s = sp[0] + sp[1] + g[...]
    c = jnp.maximum(dinv[...] * s + b[...], 0.0)
    out_o[...] = dinv[...] * _dot(c, w[...])


def _gi0_body(sp, g, dinv, b, wih, bih, gi_o):
    s = sp[0] + sp[1] + g[...]
    c = jnp.maximum(dinv[...] * s + b[...], 0.0)
    gi_o[...] = _dot(c, wih[...]) + bih[...]


def _gi1_body(yf, yr, wa, wb, bih, gi_o):
    gi_o[...] = _dot(yf[...], wa[...]) + _dot(yr[...], wb[...]) + bih[...]


def _out_body(yf, yr, wa, wb, b, o):
    o[...] = _dot(yf[...], wa[...]) + _dot(yr[...], wb[...]) + b[...]


def _row_spec(shape):
    if len(shape) == 2:
        return pl.BlockSpec((_BR, shape[1]), lambda i: (i, 0))
    return pl.BlockSpec((shape[0], _BR, shape[2]), lambda i: (0, i, 0))


def _full_spec(shape):
    return pl.BlockSpec(shape, lambda i: (0,) * len(shape))


def _blocked_call(body, n, row_ins, full_ins, out_shapes):
    grid = (n // _BR,)
    in_specs = ([_row_spec(a.shape) for a in row_ins]
                + [_full_spec(a.shape) for a in full_ins])
    out_specs = [_row_spec(s.shape) for s in out_shapes]
    fn = pl.pallas_call(body, grid=grid, in_specs=in_specs,
                        out_specs=out_specs, out_shape=out_shapes)
    return fn(*row_ins, *full_ins)


def _gru_cell(gi, gg, hprev):
    r = jax.nn.sigmoid(gi[:, 0:64] + gg[:, 0:64])
    z = jax.nn.sigmoid(gi[:, 64:128] + gg[:, 64:128])
    nv = jnp.tanh(gi[:, 128:192] + r * gg[:, 128:192])
    return (1.0 - z) * nv + z * hprev


def _rec_body(gi_ref, whhf_ref, whhr_ref, bhhf_ref, bhhr_ref,
              yf_ref, yr_ref, *, n):
    def step(t, carry):
        hf, hr = carry
        ggf = _dot(hf, whhf_ref[...]) + bhhf_ref[...]
        ggr = _dot(hr, whhr_ref[...]) + bhhr_ref[...]
        gif = gi_ref[pl.ds(t, 1), 0:192]
        gir = gi_ref[pl.ds(n - 1 - t, 1), 192:384]
        hf = _gru_cell(gif, ggf, hf)
        hr = _gru_cell(gir, ggr, hr)
        yf_ref[pl.ds(t, 1), :] = hf
        yr_ref[pl.ds(n - 1 - t, 1), :] = hr
        return (hf, hr)

    h0 = jnp.zeros((1, 64), F32)
    lax.fori_loop(0, n, step, (h0, h0))


def _gru_layer(gi, whh_f, whh_r, bhh_f, bhh_r):
    n = gi.shape[0]
    out_shapes = [jax.ShapeDtypeStruct((n, 64), F32),
                  jax.ShapeDtypeStruct((n, 64), F32)]
    fn = pl.pallas_call(functools.partial(_rec_body, n=n),
                        out_shape=out_shapes)
    return fn(gi, whh_f.T, whh_r.T, bhh_f.reshape(1, -1),
              bhh_r.reshape(1, -1))


def kernel(x, edge_index, w1, b1, w2, b2,
           w_ih_l0, w_hh_l0, b_ih_l0, b_hh_l0,
           w_ih_l0_rev, w_hh_l0_rev, b_ih_l0_rev, b_hh_l0_rev,
           w_ih_l1, w_hh_l1, b_ih_l1, b_hh_l1,
           w_ih_l1_rev, w_hh_l1_rev, b_ih_l1_rev, b_hh_l1_rev,
           w_out, b_out):
    n = x.shape[0]
    gh = w_hh_l0.shape[1]
    src = edge_index[0]
    dst = edge_index[1]

    degp = _sc_edge_scatter(jnp.ones((n, 16), F32), src, dst)
    dinv, g1 = _blocked_call(
        _pre_body, n, [degp, x], [w1],
        [jax.ShapeDtypeStruct((n, 1), F32),
         jax.ShapeDtypeStruct((n, w1.shape[1]), F32)])

    s1p = _sc_edge_scatter(g1, src, dst)
    g2, = _blocked_call(
        _mid_body, n, [s1p, g1, dinv], [b1.reshape(1, -1), w2],
        [jax.ShapeDtypeStruct((n, w2.shape[1]), F32)])

    s2p = _sc_edge_scatter(g2, src, dst)
    wih0 = jnp.concatenate([w_ih_l0.T, w_ih_l0_rev.T], axis=1)
    bih0 = jnp.concatenate([b_ih_l0, b_ih_l0_rev]).reshape(1, -1)
    gi0, = _blocked_call(
        _gi0_body, n, [s2p, g2, dinv], [b2.reshape(1, -1), wih0, bih0],
        [jax.ShapeDtypeStruct((n, 6 * gh), F32)])

    yf0, yr0 = _gru_layer(gi0, w_hh_l0, w_hh_l0_rev, b_hh_l0, b_hh_l0_rev)

    wa1 = jnp.concatenate([w_ih_l1.T[0:gh], w_ih_l1_rev.T[0:gh]], axis=1)
    wb1 = jnp.concatenate([w_ih_l1.T[gh:2 * gh], w_ih_l1_rev.T[gh:2 * gh]],
                          axis=1)
    bih1 = jnp.concatenate([b_ih_l1, b_ih_l1_rev]).reshape(1, -1)
    gi1, = _blocked_call(
        _gi1_body, n, [yf0, yr0], [wa1, wb1, bih1],
        [jax.ShapeDtypeStruct((n, 6 * gh), F32)])

    yf1, yr1 = _gru_layer(gi1, w_hh_l1, w_hh_l1_rev, b_hh_l1, b_hh_l1_rev)

    out, = _blocked_call(
        _out_body, n, [yf1, yr1],
        [w_out[0:gh], w_out[gh:2 * gh], b_out.reshape(1, -1)],
        [jax.ShapeDtypeStruct((n, w_out.shape[1]), F32)])
    return out
